# baseline (speedup 1.0000x reference)
import functools

import jax
import jax.numpy as jnp
from jax import lax
from jax.experimental import pallas as pl
from jax.experimental.pallas import tpu as pltpu
from jax.experimental.pallas import tpu_sc as plsc


def _tv_body(nv_ref, o_ref):
    x = nv_ref[...]
    o_ref[...] = jnp.sign(x) * jnp.log1p(jnp.abs(x))


_G = 128
_NBUF = 5
_DEPTH = 3

_SLAB = 64


def _make_depad(V, H):
    n_slabs = V // _SLAB
    assert n_slabs * _SLAB == V and _SLAB * H == 32 * 128
    mesh = plsc.VectorSubcoreMesh(core_axis_name="c", subcore_axis_name="s")
    per, extra = divmod(n_slabs, 32)

    @functools.partial(
        pl.kernel,
        out_type=jax.ShapeDtypeStruct((V * H // 128, 128), jnp.float32),
        mesh=mesh,
        scratch_types=[
            pltpu.VMEM((2, _SLAB, H), jnp.float32),
            pltpu.VMEM((2, 32, 128), jnp.float32),
            [pltpu.SemaphoreType.DMA] * 2,
            [pltpu.SemaphoreType.DMA] * 2,
        ],
    )
    def depad(emb_hbm, out_hbm, v1, v2, lsems, ssems):
        wid = lax.axis_index("s") * mesh.num_cores + lax.axis_index("c")
        mine = jnp.where(wid < extra, per + 1, per)
        start = wid * per + jnp.minimum(wid, extra)

        def fire_load(i, b):
            pltpu.async_copy(
                emb_hbm.at[pl.ds((start + i) * _SLAB, _SLAB), :],
                v1.at[b], lsems[b])

        def wait_load(b):
            pltpu.make_async_copy(
                emb_hbm.at[pl.ds(0, _SLAB), :], v1.at[b], lsems[b]).wait()

        def fire_store(i, b):
            pltpu.async_copy(
                v2.at[b], out_hbm.at[pl.ds((start + i) * 32, 32), :], ssems[b])

        def wait_store(b):
            pltpu.make_async_copy(
                v2.at[b], out_hbm.at[pl.ds(0, 32), :], ssems[b]).wait()

        @pl.when(mine > 0)
        def _():
            fire_load(0, 0)

        def body(j, _):
            for b in range(2):
                i = j * 2 + b

                @pl.when(i < mine)
                def _():
                    wait_load(b)

                    @pl.when(i + 1 < mine)
                    def _():
                        fire_load(i + 1, 1 - b)

                    @pl.when(i >= 2)
                    def _():
                        wait_store(b)
                    for r in range(_SLAB):
                        for c in range(H // 16):
                            v2[b, r >> 1, pl.ds((r & 1) * H + 16 * c, 16)] = (
                                v1[b, r, pl.ds(16 * c, 16)])
                    fire_store(i, b)
            return 0

        lax.fori_loop(0, (per + 2) // 2, body, 0)

        @pl.when(mine >= 2)
        def _():
            wait_store(0)
            wait_store(1)

        @pl.when(mine == 1)
        def _():
            wait_store(0)

    return depad


def _make_sc_gather(N, H, n_per_tile):
    n_chunks = n_per_tile // _G
    assert n_chunks % _NBUF == 0
    n_super = n_chunks // _NBUF
    mesh = plsc.VectorSubcoreMesh(core_axis_name="c", subcore_axis_name="s")

    @functools.partial(
        pl.kernel,
        out_type=jax.ShapeDtypeStruct((N, H), jnp.float32),
        mesh=mesh,
        scratch_types=[
            pltpu.VMEM((n_per_tile,), jnp.int32),
            pltpu.VMEM((n_per_tile,), jnp.float32),
            pltpu.VMEM((H,), jnp.float32),
            pltpu.VMEM((_NBUF, _G, H), jnp.float32),
            [pltpu.SemaphoreType.DMA] * _NBUF,
            [pltpu.SemaphoreType.DMA] * _NBUF,
        ],
        compiler_params=pltpu.CompilerParams(use_tc_tiling_on_sc=False),
    )
    def sc_gather(emb_hbm, idx_hbm, tv_hbm, dir_hbm, out_hbm,
                  idx_v, tv_v, dir_v, rows_v, gsems, osems):
        wid = lax.axis_index("s") * mesh.num_cores + lax.axis_index("c")
        tile_base = wid * n_per_tile

        pltpu.sync_copy(idx_hbm.at[pl.ds(tile_base, n_per_tile)], idx_v)
        pltpu.sync_copy(tv_hbm.at[pl.ds(tile_base, n_per_tile)], tv_v)
        pltpu.sync_copy(dir_hbm, dir_v)
        dvecs = [dir_v[pl.ds(16 * k, 16)] for k in range(H // 16)]

        def fire_gather(g, b):
            pltpu.async_copy(
                emb_hbm.at[idx_v.at[pl.ds(g * _G, _G)]], rows_v.at[b],
                gsems[b])

        def wait_gather(b):
            pltpu.make_async_copy(
                emb_hbm.at[pl.ds(0, _G)], rows_v.at[b], gsems[b]).wait()

        def fire_write(g, b):
            pltpu.async_copy(
                rows_v.at[b], out_hbm.at[pl.ds(tile_base + g * _G, _G)],
                osems[b])

        def wait_write(b):
            pltpu.make_async_copy(
                rows_v.at[b], out_hbm.at[pl.ds(0, _G)], osems[b]).wait()

        for b in range(_DEPTH):
            fire_gather(b, b)

        def super_step(s, _):
            for b in range(_NBUF):
                g = s * _NBUF + b
                wait_gather(b)
                goff = g * _G

                def grp_body(j, _):
                    tvec = tv_v[pl.ds(goff + 16 * j, 16)]
                    base = 16 * j
                    for jj in range(16):
                        t = tvec[jj]
                        i = base + jj
                        for k in range(H // 16):
                            sl = pl.ds(16 * k, 16)
                            rows_v[b, i, sl] = rows_v[b, i, sl] + t * dvecs[k]
                    return 0

                lax.fori_loop(0, _G // 16, grp_body, 0)
                fire_write(g, b)

                pb = (b + _DEPTH) % _NBUF

                @pl.when(g + _DEPTH < n_chunks)
                def _():
                    @pl.when(g >= _NBUF - _DEPTH)
                    def _():
                        wait_write(pb)
                    fire_gather(g + _DEPTH, pb)
            return 0

        lax.fori_loop(0, n_super, super_step, 0)

        for b in range(_NBUF):
            wait_write(b)

    return sc_gather


def kernel(input_ids, numerical_values, embedding, numerical_direction):
    B, S = input_ids.shape
    V, H = embedding.shape
    N = B * S
    ids = input_ids.reshape(N).astype(jnp.int32)
    nv = numerical_values.reshape(N // 128, 128)

    tv = pl.pallas_call(
        _tv_body,
        out_shape=jax.ShapeDtypeStruct((N // 128, 128), jnp.float32),
    )(nv).reshape(N)

    nw = 32
    n_per_tile = N // nw
    emb_lin = _make_depad(V, H)(embedding).reshape(V, H)
    out = _make_sc_gather(N, H, n_per_tile)(
        emb_lin, ids, tv, numerical_direction)
    return out.reshape(B, S, H)


# speedup vs baseline: 1.1074x; 1.1074x over previous
import functools

import jax
import jax.numpy as jnp
from jax import lax
from jax.experimental import pallas as pl
from jax.experimental.pallas import tpu as pltpu
from jax.experimental.pallas import tpu_sc as plsc


def _tv_body(nv_ref, o_ref):
    x = nv_ref[...]
    o_ref[...] = jnp.sign(x) * jnp.log1p(jnp.abs(x))


_G = 128
_NBUF = 5
_DEPTH = 3


def _make_sc_gather(N, B, H, n_per_tile):
    n_chunks = n_per_tile // _G
    assert n_chunks % _NBUF == 0
    n_super = n_chunks // _NBUF
    S = N // B
    CB = B // 128
    mesh = plsc.VectorSubcoreMesh(core_axis_name="c", subcore_axis_name="s")

    @functools.partial(
        pl.kernel,
        out_type=jax.ShapeDtypeStruct((S, H // 8, CB, 8, 128), jnp.float32),
        mesh=mesh,
        scratch_types=[
            pltpu.VMEM((n_per_tile,), jnp.int32),
            pltpu.VMEM((n_per_tile,), jnp.float32),
            pltpu.VMEM((H,), jnp.float32),
            pltpu.VMEM((_NBUF, _G, H), jnp.float32),
            pltpu.VMEM((2, H // 8, 8, 128), jnp.float32),
            [pltpu.SemaphoreType.DMA] * _NBUF,
            [pltpu.SemaphoreType.DMA] * 2,
        ],
        compiler_params=pltpu.CompilerParams(
            use_tc_tiling_on_sc=False, needs_layout_passes=False),
    )
    def sc_gather(emb_hbm, idx_hbm, tv_hbm, dir_hbm, out_hbm,
                  idx_v, tv_v, dir_v, rows_v, wrows_v, gsems, wsems):
        wid = lax.axis_index("s") * mesh.num_cores + lax.axis_index("c")
        tile_base = wid * n_per_tile
        blk_base = wid * n_chunks

        pltpu.sync_copy(idx_hbm.at[pl.ds(tile_base, n_per_tile)], idx_v)
        pltpu.sync_copy(tv_hbm.at[pl.ds(tile_base, n_per_tile)], tv_v)
        pltpu.sync_copy(dir_hbm, dir_v)
        dvecs = [dir_v[pl.ds(16 * k, 16)] for k in range(H // 16)]

        def fire_gather(g, b):
            pltpu.async_copy(
                emb_hbm.at[idx_v.at[pl.ds(g * _G, _G)]], rows_v.at[b],
                gsems[b])

        def wait_gather(b):
            pltpu.make_async_copy(
                emb_hbm.at[pl.ds(0, _G)], rows_v.at[b], gsems[b]).wait()

        def fire_write(g, wb):
            blk = blk_base + g
            s_i = lax.div(blk, CB)
            cb_i = lax.rem(blk, CB)
            for rh in range(H // 8):
                pltpu.async_copy(
                    wrows_v.at[wb, rh], out_hbm.at[s_i, rh, cb_i], wsems[wb])

        def wait_write(wb):
            for rh in range(H // 8):
                pltpu.make_async_copy(
                    wrows_v.at[wb, rh], out_hbm.at[0, 0, 0], wsems[wb]).wait()

        for b in range(_DEPTH):
            fire_gather(b, b)

        def super_step(s, _):
            for b in range(_NBUF):
                g = s * _NBUF + b
                wb = b % 2
                wait_gather(b)

                @pl.when(g + _DEPTH < n_chunks)
                def _():
                    fire_gather(g + _DEPTH, (b + _DEPTH) % _NBUF)

                @pl.when(g >= 2)
                def _():
                    wait_write(wb)

                goff = g * _G

                def b16_body(j, _):
                    tvec = tv_v[pl.ds(goff + 16 * j, 16)]
                    row0 = jax.lax.iota(jnp.int32, 16) + 16 * j
                    for h in range(H):
                        vals = plsc.load_gather(
                            rows_v.at[b],
                            [row0, jnp.full((16,), h, jnp.int32)])
                        t = dvecs[h // 16][h % 16]
                        wrows_v[wb, h >> 3, h & 7, pl.ds(16 * j, 16)] = (
                            vals + t * tvec)
                    return 0

                lax.fori_loop(0, _G // 16, b16_body, 0)
                fire_write(g, wb)
            return 0

        lax.fori_loop(0, n_super, super_step, 0)

        for wb in range(2):
            wait_write(wb)

    return sc_gather


def kernel(input_ids, numerical_values, embedding, numerical_direction):
    B, S = input_ids.shape
    V, H = embedding.shape
    N = B * S
    ids = input_ids.T.reshape(N).astype(jnp.int32)
    nv = numerical_values.T.reshape(N // 128, 128)

    tv = pl.pallas_call(
        _tv_body,
        out_shape=jax.ShapeDtypeStruct((N // 128, 128), jnp.float32),
    )(nv).reshape(N)

    nw = 32
    n_per_tile = N // nw
    out5 = _make_sc_gather(N, B, H, n_per_tile)(
        embedding, ids, tv, numerical_direction)
    return out5.transpose(2, 4, 0, 1, 3).reshape(B, S, H)


# speedup vs baseline: 1.3704x; 1.2375x over previous
import functools

import jax
import jax.numpy as jnp
from jax import lax
from jax.experimental import pallas as pl
from jax.experimental.pallas import tpu as pltpu
from jax.experimental.pallas import tpu_sc as plsc


def _tv_body(nv_ref, o_ref):
    x = nv_ref[...]
    o_ref[...] = jnp.sign(x) * jnp.log1p(jnp.abs(x))


_G = 128
_NBUF = 5
_DEPTH = 3


def _make_sc_gather(N, H, n_per_tile):
    n_chunks = n_per_tile // _G
    assert n_chunks % _NBUF == 0
    n_super = n_chunks // _NBUF
    mesh = plsc.VectorSubcoreMesh(core_axis_name="c", subcore_axis_name="s")

    @functools.partial(
        pl.kernel,
        out_type=jax.ShapeDtypeStruct((N, H), jnp.float32),
        mesh=mesh,
        scratch_types=[
            pltpu.VMEM((n_per_tile,), jnp.int32),
            pltpu.VMEM((n_per_tile,), jnp.float32),
            pltpu.VMEM((H,), jnp.float32),
            pltpu.VMEM((_NBUF, _G, H), jnp.float32),
            [pltpu.SemaphoreType.DMA] * _NBUF,
            [pltpu.SemaphoreType.DMA] * _NBUF,
        ],
        compiler_params=pltpu.CompilerParams(use_tc_tiling_on_sc=False),
    )
    def sc_gather(emb_hbm, idx_hbm, tv_hbm, dir_hbm, out_hbm,
                  idx_v, tv_v, dir_v, rows_v, gsems, osems):
        wid = lax.axis_index("s") * mesh.num_cores + lax.axis_index("c")
        tile_base = wid * n_per_tile

        pltpu.sync_copy(idx_hbm.at[pl.ds(tile_base, n_per_tile)], idx_v)
        pltpu.sync_copy(tv_hbm.at[pl.ds(tile_base, n_per_tile)], tv_v)
        pltpu.sync_copy(dir_hbm, dir_v)
        dvecs = [dir_v[pl.ds(16 * k, 16)] for k in range(H // 16)]

        def fire_gather(g, b):
            pltpu.async_copy(
                emb_hbm.at[idx_v.at[pl.ds(g * _G, _G)]], rows_v.at[b],
                gsems[b])

        def wait_gather(b):
            pltpu.make_async_copy(
                emb_hbm.at[pl.ds(0, _G)], rows_v.at[b], gsems[b]).wait()

        def fire_write(g, b):
            pltpu.async_copy(
                rows_v.at[b], out_hbm.at[pl.ds(tile_base + g * _G, _G)],
                osems[b])

        def wait_write(b):
            pltpu.make_async_copy(
                rows_v.at[b], out_hbm.at[pl.ds(0, _G)], osems[b]).wait()

        for b in range(_DEPTH):
            fire_gather(b, b)

        def super_step(s, _):
            for b in range(_NBUF):
                g = s * _NBUF + b
                wait_gather(b)
                goff = g * _G

                def grp_body(j, _):
                    tvec = tv_v[pl.ds(goff + 16 * j, 16)]
                    base = 16 * j
                    for jj in range(16):
                        t = tvec[jj]
                        i = base + jj
                        for k in range(H // 16):
                            sl = pl.ds(16 * k, 16)
                            rows_v[b, i, sl] = rows_v[b, i, sl] + t * dvecs[k]
                    return 0

                lax.fori_loop(0, _G // 16, grp_body, 0)
                fire_write(g, b)

                pb = (b + _DEPTH) % _NBUF

                @pl.when(g + _DEPTH < n_chunks)
                def _():
                    @pl.when(g >= _NBUF - _DEPTH)
                    def _():
                        wait_write(pb)
                    fire_gather(g + _DEPTH, pb)
            return 0

        lax.fori_loop(0, n_super, super_step, 0)

        for b in range(_NBUF):
            wait_write(b)

    return sc_gather


def kernel(input_ids, numerical_values, embedding, numerical_direction):
    B, S = input_ids.shape
    V, H = embedding.shape
    N = B * S
    ids = input_ids.reshape(N).astype(jnp.int32)
    nv = numerical_values.reshape(N // 128, 128)

    tv = pl.pallas_call(
        _tv_body,
        out_shape=jax.ShapeDtypeStruct((N // 128, 128), jnp.float32),
    )(nv).reshape(N)

    nw = 32
    n_per_tile = N // nw
    out = _make_sc_gather(N, H, n_per_tile)(
        embedding, ids, tv, numerical_direction)
    return out.reshape(B, S, H)
